# baseline (speedup 1.0000x reference)
import functools

import jax
import jax.numpy as jnp
from jax import lax
from jax.experimental import pallas as pl
from jax.experimental.pallas import tpu as pltpu
from jax.experimental.pallas import tpu_sc as plsc

NC = 2
NS = 16
NW = NC * NS
L = 16
CHUNK = 128


def _mesh():
    return plsc.VectorSubcoreMesh(
        core_axis_name="c", subcore_axis_name="s",
        num_cores=NC, num_subcores=NS)


def _zero_rows(buf, acc_sh, r0, rpt):
    n_full = rpt // CHUNK
    rem = rpt % CHUNK
    for k in range(n_full):
        pltpu.sync_copy(buf, acc_sh.at[pl.ds(r0 + k * CHUNK, CHUNK)])
    if rem:
        pltpu.sync_copy(buf.at[pl.ds(0, rem)],
                        acc_sh.at[pl.ds(r0 + n_full * CHUNK, rem)])


def _sc_degree(dst2, n_pad, nchunk):
    @functools.partial(
        pl.kernel,
        out_type=jax.ShapeDtypeStruct((NC * n_pad, L), jnp.float32),
        mesh=_mesh(),
        compiler_params=pltpu.CompilerParams(use_tc_tiling_on_sc=False),
        scratch_types=[
            pltpu.VMEM((nchunk, CHUNK), jnp.int32),
            pltpu.VMEM((CHUNK, L), jnp.float32),
            pltpu.VMEM_SHARED((n_pad, L), jnp.float32),
        ],
    )
    def deg_kernel(dst_hbm, out_hbm, dst_v, ones_v, acc_sh):
        c = lax.axis_index("c")
        s = lax.axis_index("s")
        wid = c * NS + s
        rpt = n_pad // NS
        r0 = s * rpt

        @pl.loop(0, CHUNK)
        def _(r):
            ones_v[r, pl.ds(0, L)] = jnp.zeros((L,), jnp.float32)

        _zero_rows(ones_v, acc_sh, r0, rpt)

        @pl.loop(0, CHUNK)
        def _(r):
            ones_v[r, pl.ds(0, L)] = jnp.full((L,), 1.0, jnp.float32)

        pltpu.sync_copy(dst_hbm.at[pl.ds(wid * nchunk, nchunk)], dst_v)
        plsc.subcore_barrier()

        @pl.loop(0, nchunk)
        def _(k):
            pltpu.sync_copy(ones_v, acc_sh.at[dst_v.at[k]], add=True)

        plsc.subcore_barrier()
        pltpu.sync_copy(acc_sh.at[pl.ds(r0, rpt)],
                        out_hbm.at[pl.ds(c * n_pad + r0, rpt)])

    return deg_kernel(dst2)


def _sc_agg(hs, sd2, n_pad, nchunk):
    d_dim = hs.shape[1]

    @functools.partial(
        pl.kernel,
        out_type=jax.ShapeDtypeStruct((NC * n_pad, d_dim), jnp.float32),
        mesh=_mesh(),
        compiler_params=pltpu.CompilerParams(use_tc_tiling_on_sc=False),
        scratch_types=[
            pltpu.VMEM((2, 2, CHUNK), jnp.int32),
            pltpu.VMEM((2, 2, CHUNK), jnp.int32),
            pltpu.VMEM((CHUNK, d_dim), jnp.float32),
            pltpu.VMEM((CHUNK, d_dim), jnp.float32),
            pltpu.VMEM_SHARED((n_pad, d_dim), jnp.float32),
            pltpu.SemaphoreType.DMA,
            pltpu.SemaphoreType.DMA,
            pltpu.SemaphoreType.DMA,
            pltpu.SemaphoreType.DMA,
            pltpu.SemaphoreType.DMA,
            pltpu.SemaphoreType.DMA,
        ],
    )
    def agg_kernel(hs_hbm, sd_hbm, out_hbm,
                   idx_A, idx_B, rows_a, rows_b, acc_sh,
                   sem_a, sem_b, sem_ia, sem_ib, sem_sa, sem_sb):
        c = lax.axis_index("c")
        s = lax.axis_index("s")
        wid = c * NS + s
        rpt = n_pad // NS
        r0 = s * rpt

        @pl.loop(0, CHUNK)
        def _(r):
            @pl.loop(0, d_dim, step=L)
            def _(j):
                rows_a[r, pl.ds(j, L)] = jnp.zeros((L,), jnp.float32)

        _zero_rows(rows_a, acc_sh, r0, rpt)
        plsc.subcore_barrier()

        k0 = wid * nchunk

        def wait_g(buf, sem):
            pltpu.make_async_copy(hs_hbm.at[idx_A.at[0, 0]], buf, sem).wait()

        def wait_i(idx, sem):
            pltpu.make_async_copy(sd_hbm.at[pl.ds(k0, 2)], idx, sem).wait()

        def wait_s(buf, sem):
            pltpu.make_async_copy(buf, acc_sh.at[idx_A.at[0, 1]], sem).wait()

        pltpu.sync_copy(sd_hbm.at[pl.ds(k0, 2)], idx_A)
        pltpu.async_copy(sd_hbm.at[pl.ds(k0 + 2, 2)], idx_B, sem_ib)
        pltpu.async_copy(hs_hbm.at[idx_A.at[0, 0]], rows_a, sem_a)
        pltpu.async_copy(hs_hbm.at[idx_A.at[1, 0]], rows_b, sem_b)

        @pl.loop(0, nchunk - 4, step=4)
        def _(k):
            wait_g(rows_a, sem_a)
            pltpu.async_copy(rows_a, acc_sh.at[idx_A.at[0, 1]], sem_sa,
                             add=True)
            wait_g(rows_b, sem_b)
            pltpu.async_copy(rows_b, acc_sh.at[idx_A.at[1, 1]], sem_sb,
                             add=True)
            wait_i(idx_B, sem_ib)
            wait_s(rows_a, sem_sa)
            pltpu.async_copy(hs_hbm.at[idx_B.at[0, 0]], rows_a, sem_a)
            pltpu.async_copy(sd_hbm.at[pl.ds(k0 + k + 4, 2)], idx_A, sem_ia)
            wait_s(rows_b, sem_sb)
            pltpu.async_copy(hs_hbm.at[idx_B.at[1, 0]], rows_b, sem_b)

            wait_g(rows_a, sem_a)
            pltpu.async_copy(rows_a, acc_sh.at[idx_B.at[0, 1]], sem_sa,
                             add=True)
            wait_g(rows_b, sem_b)
            pltpu.async_copy(rows_b, acc_sh.at[idx_B.at[1, 1]], sem_sb,
                             add=True)
            wait_i(idx_A, sem_ia)
            wait_s(rows_a, sem_sa)
            pltpu.async_copy(hs_hbm.at[idx_A.at[0, 0]], rows_a, sem_a)
            pltpu.async_copy(sd_hbm.at[pl.ds(k0 + k + 6, 2)], idx_B, sem_ib)
            wait_s(rows_b, sem_sb)
            pltpu.async_copy(hs_hbm.at[idx_A.at[1, 0]], rows_b, sem_b)

        wait_g(rows_a, sem_a)
        pltpu.sync_copy(rows_a, acc_sh.at[idx_A.at[0, 1]], add=True)
        wait_i(idx_B, sem_ib)
        pltpu.async_copy(hs_hbm.at[idx_B.at[0, 0]], rows_a, sem_a)
        wait_g(rows_b, sem_b)
        pltpu.sync_copy(rows_b, acc_sh.at[idx_A.at[1, 1]], add=True)
        pltpu.async_copy(hs_hbm.at[idx_B.at[1, 0]], rows_b, sem_b)
        wait_g(rows_a, sem_a)
        pltpu.sync_copy(rows_a, acc_sh.at[idx_B.at[0, 1]], add=True)
        wait_g(rows_b, sem_b)
        pltpu.sync_copy(rows_b, acc_sh.at[idx_B.at[1, 1]], add=True)

        plsc.subcore_barrier()
        pltpu.sync_copy(acc_sh.at[pl.ds(r0, rpt)],
                        out_hbm.at[pl.ds(c * n_pad + r0, rpt)])

    return agg_kernel(hs, sd2)


BN = 632


def _mm_scale(x_pad, w, degp):
    n_pad, k = x_pad.shape
    m = w.shape[1]

    def body(x_ref, w_ref, d0_ref, d1_ref, o_ref):
        h = jnp.dot(x_ref[...], w_ref[...], preferred_element_type=jnp.float32)
        o_ref[...] = h * _dis_block(d0_ref, d1_ref)

    return pl.pallas_call(
        body,
        grid=(n_pad // BN,),
        in_specs=[pl.BlockSpec((BN, k), lambda i: (i, 0)),
                  pl.BlockSpec((k, m), lambda i: (0, 0)),
                  pl.BlockSpec((BN, L), lambda i: (i, 0)),
                  pl.BlockSpec((BN, L), lambda i: (16 + i, 0))],
        out_specs=pl.BlockSpec((BN, m), lambda i: (i, 0)),
        out_shape=jax.ShapeDtypeStruct((n_pad, m), jnp.float32),
    )(x_pad, w, degp, degp)


def _dis_block(d0_ref, d1_ref):
    return lax.rsqrt(d0_ref[:, 0:1] + d1_ref[:, 0:1] + 1.0)


def _combine_mm(a1, hs1, degp, b1, w2):
    n_pad, m = hs1.shape
    m2 = w2.shape[1]

    def body(p0_ref, p1_ref, hs_ref, d0_ref, d1_ref, b_ref, w_ref, o_ref):
        dis = _dis_block(d0_ref, d1_ref)
        pre = dis * (p0_ref[...] + p1_ref[...] + hs_ref[...]) + b_ref[...]
        h2 = jnp.maximum(pre, 0.0)
        o_ref[...] = dis * jnp.dot(h2, w_ref[...],
                                   preferred_element_type=jnp.float32)

    return pl.pallas_call(
        body,
        grid=(n_pad // BN,),
        in_specs=[pl.BlockSpec((BN, m), lambda i: (i, 0)),
                  pl.BlockSpec((BN, m), lambda i: (16 + i, 0)),
                  pl.BlockSpec((BN, m), lambda i: (i, 0)),
                  pl.BlockSpec((BN, L), lambda i: (i, 0)),
                  pl.BlockSpec((BN, L), lambda i: (16 + i, 0)),
                  pl.BlockSpec((1, m), lambda i: (0, 0)),
                  pl.BlockSpec((m, m2), lambda i: (0, 0))],
        out_specs=pl.BlockSpec((BN, m2), lambda i: (i, 0)),
        out_shape=jax.ShapeDtypeStruct((n_pad, m2), jnp.float32),
    )(a1, a1, hs1, degp, degp, b1.reshape(1, m), w2)


def _final(a2, hs2, degp, b2):
    n_pad, m = hs2.shape

    def body(p0_ref, p1_ref, hs_ref, d0_ref, d1_ref, b_ref, o_ref):
        dis = _dis_block(d0_ref, d1_ref)
        o_ref[...] = dis * (p0_ref[...] + p1_ref[...] + hs_ref[...]) + b_ref[...]

    return pl.pallas_call(
        body,
        grid=(n_pad // BN,),
        in_specs=[pl.BlockSpec((BN, m), lambda i: (i, 0)),
                  pl.BlockSpec((BN, m), lambda i: (16 + i, 0)),
                  pl.BlockSpec((BN, m), lambda i: (i, 0)),
                  pl.BlockSpec((BN, L), lambda i: (i, 0)),
                  pl.BlockSpec((BN, L), lambda i: (16 + i, 0)),
                  pl.BlockSpec((1, m), lambda i: (0, 0))],
        out_specs=pl.BlockSpec((BN, m), lambda i: (i, 0)),
        out_shape=jax.ShapeDtypeStruct((n_pad, m), jnp.float32),
    )(a2, a2, hs2, degp, degp, b2.reshape(1, m))


def kernel(x, edge_index, W1, b1, W2, b2):
    n = x.shape[0]
    e = edge_index.shape[1]
    src = edge_index[0].astype(jnp.int32)
    dst = edge_index[1].astype(jnp.int32)

    nt = -(-e // NW)
    nt = -(-nt // (4 * CHUNK)) * 4 * CHUNK
    nchunk = nt // CHUNK
    e_pad = nt * NW
    n_pad = (n // (NS * 8) + 1) * NS * 8

    pad = e_pad - e
    pad_iota = jnp.arange(pad, dtype=jnp.int32)
    src_p = jnp.concatenate([src, pad_iota % n])
    dst_p = jnp.concatenate([dst, n + pad_iota % (n_pad - n)])
    dst2 = dst_p.reshape(NW * nchunk, CHUNK)
    sd2 = jnp.stack([src_p.reshape(NW * nchunk, CHUNK), dst2], axis=1)

    x_pad = jnp.concatenate(
        [x, jnp.zeros((n_pad - n, x.shape[1]), jnp.float32)])

    degp = _sc_degree(dst2, n_pad, nchunk)
    hs1 = _mm_scale(x_pad, W1, degp)
    a1 = _sc_agg(hs1, sd2, n_pad, nchunk)
    hs2 = _combine_mm(a1, hs1, degp, b1, W2)
    a2 = _sc_agg(hs2, sd2, n_pad, nchunk)
    return _final(a2, hs2, degp, b2)[:n]


# speedup vs baseline: 1.1855x; 1.1855x over previous
import functools

import jax
import jax.numpy as jnp
from jax import lax
from jax.experimental import pallas as pl
from jax.experimental.pallas import tpu as pltpu
from jax.experimental.pallas import tpu_sc as plsc

NC = 2
NS = 16
NW = NC * NS
L = 16
CHUNK = 128


def _mesh():
    return plsc.VectorSubcoreMesh(
        core_axis_name="c", subcore_axis_name="s",
        num_cores=NC, num_subcores=NS)


def _zero_rows(buf, acc_sh, r0, rpt):
    n_full = rpt // CHUNK
    rem = rpt % CHUNK
    for k in range(n_full):
        pltpu.sync_copy(buf, acc_sh.at[pl.ds(r0 + k * CHUNK, CHUNK)])
    if rem:
        pltpu.sync_copy(buf.at[pl.ds(0, rem)],
                        acc_sh.at[pl.ds(r0 + n_full * CHUNK, rem)])


def _sc_degree(dst2, n_pad, nchunk):
    @functools.partial(
        pl.kernel,
        out_type=jax.ShapeDtypeStruct((NC * n_pad, L), jnp.float32),
        mesh=_mesh(),
        compiler_params=pltpu.CompilerParams(use_tc_tiling_on_sc=False),
        scratch_types=[
            pltpu.VMEM((nchunk, CHUNK), jnp.int32),
            pltpu.VMEM((CHUNK, L), jnp.float32),
            pltpu.VMEM_SHARED((n_pad, L), jnp.float32),
        ],
    )
    def deg_kernel(dst_hbm, out_hbm, dst_v, ones_v, acc_sh):
        c = lax.axis_index("c")
        s = lax.axis_index("s")
        wid = c * NS + s
        rpt = n_pad // NS
        r0 = s * rpt

        @pl.loop(0, CHUNK)
        def _(r):
            ones_v[r, pl.ds(0, L)] = jnp.zeros((L,), jnp.float32)

        _zero_rows(ones_v, acc_sh, r0, rpt)

        @pl.loop(0, CHUNK)
        def _(r):
            ones_v[r, pl.ds(0, L)] = jnp.full((L,), 1.0, jnp.float32)

        pltpu.sync_copy(dst_hbm.at[pl.ds(wid * nchunk, nchunk)], dst_v)
        plsc.subcore_barrier()

        @pl.loop(0, nchunk)
        def _(k):
            pltpu.sync_copy(ones_v, acc_sh.at[dst_v.at[k]], add=True)

        plsc.subcore_barrier()
        pltpu.sync_copy(acc_sh.at[pl.ds(r0, rpt)],
                        out_hbm.at[pl.ds(c * n_pad + r0, rpt)])

    return deg_kernel(dst2)


def _sc_agg(hs, sd2, n_pad, nchunk):
    d_dim = hs.shape[1]

    @functools.partial(
        pl.kernel,
        out_type=jax.ShapeDtypeStruct((NC * n_pad, d_dim), jnp.float32),
        mesh=_mesh(),
        compiler_params=pltpu.CompilerParams(use_tc_tiling_on_sc=False),
        scratch_types=[
            pltpu.VMEM((2, 2, CHUNK), jnp.int32),
            pltpu.VMEM((2, 2, CHUNK), jnp.int32),
            pltpu.VMEM((CHUNK, d_dim), jnp.float32),
            pltpu.VMEM((CHUNK, d_dim), jnp.float32),
            pltpu.VMEM_SHARED((n_pad, d_dim), jnp.float32),
            pltpu.SemaphoreType.DMA,
            pltpu.SemaphoreType.DMA,
            pltpu.SemaphoreType.DMA,
            pltpu.SemaphoreType.DMA,
        ],
    )
    def agg_kernel(hs_hbm, sd_hbm, out_hbm,
                   idx_A, idx_B, rows_a, rows_b, acc_sh,
                   sem_a, sem_b, sem_ia, sem_ib):
        c = lax.axis_index("c")
        s = lax.axis_index("s")
        wid = c * NS + s
        rpt = n_pad // NS
        r0 = s * rpt

        @pl.loop(0, CHUNK)
        def _(r):
            @pl.loop(0, d_dim, step=L)
            def _(j):
                rows_a[r, pl.ds(j, L)] = jnp.zeros((L,), jnp.float32)

        _zero_rows(rows_a, acc_sh, r0, rpt)
        plsc.subcore_barrier()

        k0 = wid * nchunk

        def wait_g(buf, sem):
            pltpu.make_async_copy(hs_hbm.at[idx_A.at[0, 0]], buf, sem).wait()

        def wait_i(idx, sem):
            pltpu.make_async_copy(sd_hbm.at[pl.ds(k0, 2)], idx, sem).wait()

        pltpu.sync_copy(sd_hbm.at[pl.ds(k0, 2)], idx_A)
        pltpu.async_copy(sd_hbm.at[pl.ds(k0 + 2, 2)], idx_B, sem_ib)
        pltpu.async_copy(hs_hbm.at[idx_A.at[0, 0]], rows_a, sem_a)
        pltpu.async_copy(hs_hbm.at[idx_A.at[1, 0]], rows_b, sem_b)

        @pl.loop(0, nchunk - 4, step=4)
        def _(k):
            wait_g(rows_a, sem_a)
            pltpu.sync_copy(rows_a, acc_sh.at[idx_A.at[0, 1]], add=True)
            wait_i(idx_B, sem_ib)
            pltpu.async_copy(hs_hbm.at[idx_B.at[0, 0]], rows_a, sem_a)
            wait_g(rows_b, sem_b)
            pltpu.sync_copy(rows_b, acc_sh.at[idx_A.at[1, 1]], add=True)
            pltpu.async_copy(sd_hbm.at[pl.ds(k0 + k + 4, 2)], idx_A, sem_ia)
            pltpu.async_copy(hs_hbm.at[idx_B.at[1, 0]], rows_b, sem_b)

            wait_g(rows_a, sem_a)
            pltpu.sync_copy(rows_a, acc_sh.at[idx_B.at[0, 1]], add=True)
            wait_i(idx_A, sem_ia)
            pltpu.async_copy(hs_hbm.at[idx_A.at[0, 0]], rows_a, sem_a)
            wait_g(rows_b, sem_b)
            pltpu.sync_copy(rows_b, acc_sh.at[idx_B.at[1, 1]], add=True)
            pltpu.async_copy(sd_hbm.at[pl.ds(k0 + k + 6, 2)], idx_B, sem_ib)
            pltpu.async_copy(hs_hbm.at[idx_A.at[1, 0]], rows_b, sem_b)

        wait_g(rows_a, sem_a)
        pltpu.sync_copy(rows_a, acc_sh.at[idx_A.at[0, 1]], add=True)
        wait_i(idx_B, sem_ib)
        pltpu.async_copy(hs_hbm.at[idx_B.at[0, 0]], rows_a, sem_a)
        wait_g(rows_b, sem_b)
        pltpu.sync_copy(rows_b, acc_sh.at[idx_A.at[1, 1]], add=True)
        pltpu.async_copy(hs_hbm.at[idx_B.at[1, 0]], rows_b, sem_b)
        wait_g(rows_a, sem_a)
        pltpu.sync_copy(rows_a, acc_sh.at[idx_B.at[0, 1]], add=True)
        wait_g(rows_b, sem_b)
        pltpu.sync_copy(rows_b, acc_sh.at[idx_B.at[1, 1]], add=True)

        plsc.subcore_barrier()
        pltpu.sync_copy(acc_sh.at[pl.ds(r0, rpt)],
                        out_hbm.at[pl.ds(c * n_pad + r0, rpt)])

    return agg_kernel(hs, sd2)


BN = 632


def _mm_scale(x_pad, w, degp):
    n_pad, k = x_pad.shape
    m = w.shape[1]

    def body(x_ref, w_ref, d0_ref, d1_ref, o_ref):
        h = jnp.dot(x_ref[...], w_ref[...], preferred_element_type=jnp.float32)
        o_ref[...] = h * _dis_block(d0_ref, d1_ref)

    return pl.pallas_call(
        body,
        grid=(n_pad // BN,),
        in_specs=[pl.BlockSpec((BN, k), lambda i: (i, 0)),
                  pl.BlockSpec((k, m), lambda i: (0, 0)),
                  pl.BlockSpec((BN, L), lambda i: (i, 0)),
                  pl.BlockSpec((BN, L), lambda i: (16 + i, 0))],
        out_specs=pl.BlockSpec((BN, m), lambda i: (i, 0)),
        out_shape=jax.ShapeDtypeStruct((n_pad, m), jnp.float32),
    )(x_pad, w, degp, degp)


def _dis_block(d0_ref, d1_ref):
    return lax.rsqrt(d0_ref[:, 0:1] + d1_ref[:, 0:1] + 1.0)


def _combine_mm(a1, hs1, degp, b1, w2):
    n_pad, m = hs1.shape
    m2 = w2.shape[1]

    def body(p0_ref, p1_ref, hs_ref, d0_ref, d1_ref, b_ref, w_ref, o_ref):
        dis = _dis_block(d0_ref, d1_ref)
        pre = dis * (p0_ref[...] + p1_ref[...] + hs_ref[...]) + b_ref[...]
        h2 = jnp.maximum(pre, 0.0)
        o_ref[...] = dis * jnp.dot(h2, w_ref[...],
                                   preferred_element_type=jnp.float32)

    return pl.pallas_call(
        body,
        grid=(n_pad // BN,),
        in_specs=[pl.BlockSpec((BN, m), lambda i: (i, 0)),
                  pl.BlockSpec((BN, m), lambda i: (16 + i, 0)),
                  pl.BlockSpec((BN, m), lambda i: (i, 0)),
                  pl.BlockSpec((BN, L), lambda i: (i, 0)),
                  pl.BlockSpec((BN, L), lambda i: (16 + i, 0)),
                  pl.BlockSpec((1, m), lambda i: (0, 0)),
                  pl.BlockSpec((m, m2), lambda i: (0, 0))],
        out_specs=pl.BlockSpec((BN, m2), lambda i: (i, 0)),
        out_shape=jax.ShapeDtypeStruct((n_pad, m2), jnp.float32),
    )(a1, a1, hs1, degp, degp, b1.reshape(1, m), w2)


def _final(a2, hs2, degp, b2):
    n_pad, m = hs2.shape

    def body(p0_ref, p1_ref, hs_ref, d0_ref, d1_ref, b_ref, o_ref):
        dis = _dis_block(d0_ref, d1_ref)
        o_ref[...] = dis * (p0_ref[...] + p1_ref[...] + hs_ref[...]) + b_ref[...]

    return pl.pallas_call(
        body,
        grid=(n_pad // BN,),
        in_specs=[pl.BlockSpec((BN, m), lambda i: (i, 0)),
                  pl.BlockSpec((BN, m), lambda i: (16 + i, 0)),
                  pl.BlockSpec((BN, m), lambda i: (i, 0)),
                  pl.BlockSpec((BN, L), lambda i: (i, 0)),
                  pl.BlockSpec((BN, L), lambda i: (16 + i, 0)),
                  pl.BlockSpec((1, m), lambda i: (0, 0))],
        out_specs=pl.BlockSpec((BN, m), lambda i: (i, 0)),
        out_shape=jax.ShapeDtypeStruct((n_pad, m), jnp.float32),
    )(a2, a2, hs2, degp, degp, b2.reshape(1, m))


def kernel(x, edge_index, W1, b1, W2, b2):
    n = x.shape[0]
    e = edge_index.shape[1]
    src = edge_index[0].astype(jnp.int32)
    dst = edge_index[1].astype(jnp.int32)

    nt = -(-e // NW)
    nt = -(-nt // (4 * CHUNK)) * 4 * CHUNK
    nchunk = nt // CHUNK
    e_pad = nt * NW
    n_pad = (n // (NS * 8) + 1) * NS * 8

    pad = e_pad - e
    pad_iota = jnp.arange(pad, dtype=jnp.int32)
    src_p = jnp.concatenate([src, pad_iota % n])
    dst_p = jnp.concatenate([dst, n + pad_iota % (n_pad - n)])
    dst2 = dst_p.reshape(NW * nchunk, CHUNK)
    sd2 = jnp.stack([src_p.reshape(NW * nchunk, CHUNK), dst2], axis=1)

    x_pad = jnp.concatenate(
        [x, jnp.zeros((n_pad - n, x.shape[1]), jnp.float32)])

    degp = _sc_degree(dst2, n_pad, nchunk)
    hs1 = _mm_scale(x_pad, W1, degp)
    a1 = _sc_agg(hs1, sd2, n_pad, nchunk)
    hs2 = _combine_mm(a1, hs1, degp, b1, W2)
    a2 = _sc_agg(hs2, sd2, n_pad, nchunk)
    return _final(a2, hs2, degp, b2)[:n]
